# baseline (speedup 1.0000x reference)
import functools

import jax
import jax.numpy as jnp
from jax import lax
from jax.experimental import pallas as pl
from jax.experimental.pallas import tpu as pltpu
from jax.experimental.pallas import tpu_sc as plsc

B, C, S, H = 1024, 12, 50, 64
NC, NS = 2, 16
NW = NC * NS
P = B * S
PW = P // NW
RW = PW * C
CP = 8
CR = CP * C
NSTEP = PW // CP
NPOS = C * S


def _sc_fused(idx, word_table, pos_table, chw16, gamma, beta):
    mesh = plsc.VectorSubcoreMesh(core_axis_name="c", subcore_axis_name="s")

    @functools.partial(
        pl.kernel,
        out_type=jax.ShapeDtypeStruct((P, H), jnp.float32),
        mesh=mesh,
        compiler_params=pltpu.CompilerParams(needs_layout_passes=False,
                                             use_tc_tiling_on_sc=False),
        scratch_types=[
            pltpu.VMEM((NPOS, H), jnp.float32),
            pltpu.VMEM((H,), jnp.float32),
            pltpu.VMEM((H,), jnp.float32),
            pltpu.VMEM((16,), jnp.float32),
            pltpu.SMEM((16,), jnp.float32),
            pltpu.VMEM((RW,), jnp.int32),
            pltpu.VMEM((CR, H), jnp.float32),
            pltpu.VMEM((CR, H), jnp.float32),
            pltpu.VMEM((CP, H), jnp.float32),
            pltpu.VMEM((CP, H), jnp.float32),
            pltpu.SemaphoreType.DMA,
            pltpu.SemaphoreType.DMA,
            pltpu.SemaphoreType.DMA,
            pltpu.SemaphoreType.DMA,
        ],
    )
    def k(idx_hbm, wt_hbm, pos_hbm, chw_hbm, g_hbm, b_hbm, out_hbm,
          posv, gv, bv, wv, wsm, idxv, rb0, rb1,
          ob0, ob1, sem0, sem1, osem0, osem1):
        wid = lax.axis_index("s") * NC + lax.axis_index("c")
        row0 = wid * RW
        pair0 = wid * PW

        pltpu.sync_copy(pos_hbm, posv)
        pltpu.sync_copy(g_hbm, gv)
        pltpu.sync_copy(b_hbm, bv)
        pltpu.sync_copy(chw_hbm.at[pl.ds(0, 16)], wv)
        wvec = wv[...]
        for i in range(C):
            wsm[i] = wvec[i]

        pltpu.sync_copy(idx_hbm.at[pl.ds(row0, RW)], idxv)

        def gat(g, rb, sem):
            off = pl.multiple_of(g * CR, 8)
            return pltpu.make_async_copy(wt_hbm.at[idxv.at[pl.ds(off, CR)]],
                                         rb, sem)

        def owrite(g, ob, osem):
            off = pl.multiple_of(pair0 + g * CP, 8)
            return pltpu.make_async_copy(ob, out_hbm.at[pl.ds(off, CP)], osem)

        def process(g, rb, ob):
            gr0 = row0 + g * CR

            def body_p(p2, carry):
                for u in range(2):
                    p = p2 * 2 + u
                    base = p * C
                    acc = [jnp.zeros((16,), jnp.float32) for _ in range(4)]
                    offs = jnp.float32(0.0)
                    for c in range(C):
                        r = base + c
                        prow = lax.rem(gr0 + r, NPOS)
                        e = []
                        for kk in range(4):
                            e.append(rb[r, pl.ds(16 * kk, 16)]
                                     + posv[prow, pl.ds(16 * kk, 16)])
                        s = jnp.sum((e[0] + e[1]) + (e[2] + e[3]), axis=0)
                        q = jnp.sum((e[0] * e[0] + e[1] * e[1])
                                    + (e[2] * e[2] + e[3] * e[3]), axis=0)
                        mu = s * (1.0 / H)
                        var = q * (1.0 / H) - mu * mu + 1e-5
                        bi = lax.bitcast_convert_type(var, jnp.int32)
                        y = lax.bitcast_convert_type(
                            jnp.int32(0x5F3759DF) - (bi >> 1), jnp.float32)
                        for _ in range(4):
                            y = y * (1.5 - (0.5 * var) * (y * y))
                        a = wsm[c] * y
                        offs = offs + a * mu
                        for kk in range(4):
                            acc[kk] = acc[kk] + a * e[kk]
                    for kk in range(4):
                        ob[p, pl.ds(16 * kk, 16)] = (
                            (acc[kk] - offs) * gv[pl.ds(16 * kk, 16)]
                            + bv[pl.ds(16 * kk, 16)])
                return carry

            lax.fori_loop(0, CP // 2, body_p, 0)

        gat(0, rb0, sem0).start()

        def body_m(h, carry):
            g0 = h * 2
            gat(g0, rb0, sem0).wait()
            gat(g0 + 1, rb1, sem1).start()

            @pl.when(g0 >= 2)
            def _():
                owrite(g0 - 2, ob0, osem0).wait()

            process(g0, rb0, ob0)
            owrite(g0, ob0, osem0).start()

            gat(g0 + 1, rb1, sem1).wait()

            @pl.when(g0 + 2 < NSTEP)
            def _():
                gat(g0 + 2, rb0, sem0).start()

            @pl.when(g0 >= 1)
            def _():
                owrite(g0 - 1, ob1, osem1).wait()

            process(g0 + 1, rb1, ob1)
            owrite(g0 + 1, ob1, osem1).start()
            return carry

        lax.fori_loop(0, NSTEP // 2, body_m, 0)
        owrite(NSTEP - 2, ob0, osem0).wait()
        owrite(NSTEP - 1, ob1, osem1).wait()

    return k(idx, word_table, pos_table, chw16, gamma, beta)


def _tc_softmax(chw128):
    def body(cb, ob):
        v = cb[...]
        lanei = lax.broadcasted_iota(jnp.int32, (1, 128), 1)
        valid = lanei < C
        vm = jnp.where(valid, v, -1e30)
        m = jnp.max(vm, axis=1, keepdims=True)
        e = jnp.where(valid, jnp.exp(vm - m), 0.0)
        ob[...] = e / jnp.sum(e, axis=1, keepdims=True)

    return pl.pallas_call(
        body,
        out_shape=jax.ShapeDtypeStruct((1, 128), jnp.float32),
    )(chw128)


def _tc_head(fused, w, b2d, g2d, be2d):
    blk = 512

    def body(xb, wb, bb, gb, beb, ob):
        y = jnp.dot(xb[...], wb[...], preferred_element_type=jnp.float32)
        y = y + bb[...]
        mu = jnp.mean(y, axis=1, keepdims=True)
        var = jnp.mean((y - mu) ** 2, axis=1, keepdims=True)
        ob[...] = (y - mu) * lax.rsqrt(var + 1e-5) * gb[...] + beb[...]

    return pl.pallas_call(
        body,
        grid=(P // blk,),
        in_specs=[
            pl.BlockSpec((blk, H), lambda i: (i, 0)),
            pl.BlockSpec((H, H), lambda i: (0, 0)),
            pl.BlockSpec((1, H), lambda i: (0, 0)),
            pl.BlockSpec((1, H), lambda i: (0, 0)),
            pl.BlockSpec((1, H), lambda i: (0, 0)),
        ],
        out_specs=pl.BlockSpec((blk, H), lambda i: (i, 0)),
        out_shape=jax.ShapeDtypeStruct((P, H), jnp.float32),
    )(fused, w, b2d, g2d, be2d)


def kernel(x, word_table, pos_table, gamma, beta, ch_w, W, b_lin, gamma2, beta2):
    idx = jnp.transpose(x, (0, 2, 1)).reshape(P * C)
    w128 = _tc_softmax(jnp.pad(ch_w, (0, 128 - C)).reshape(1, 128))
    fused = _sc_fused(idx, word_table, pos_table, w128.reshape(128),
                      gamma, beta)
    out = _tc_head(fused, W, b_lin.reshape(1, H), gamma2.reshape(1, H),
                   beta2.reshape(1, H))
    return out.reshape(B, S, H)


# speedup vs baseline: 1.3137x; 1.3137x over previous
import functools

import jax
import jax.numpy as jnp
from jax import lax
from jax.experimental import pallas as pl
from jax.experimental.pallas import tpu as pltpu
from jax.experimental.pallas import tpu_sc as plsc

B, C, S, H = 1024, 12, 50, 64
NC, NS = 2, 16
NW = NC * NS
P = B * S
PW = P // NW
RW = PW * C
CP = 8
CR = CP * C
NSTEP = PW // CP
NPOS = C * S


def _sc_fused(idx, word_table, pos_table, chw16, gamma, beta):
    mesh = plsc.VectorSubcoreMesh(core_axis_name="c", subcore_axis_name="s")

    @functools.partial(
        pl.kernel,
        out_type=jax.ShapeDtypeStruct((P, H), jnp.float32),
        mesh=mesh,
        compiler_params=pltpu.CompilerParams(needs_layout_passes=False,
                                             use_tc_tiling_on_sc=False),
        scratch_types=[
            pltpu.VMEM((NPOS, H), jnp.float32),
            pltpu.VMEM((H,), jnp.float32),
            pltpu.VMEM((H,), jnp.float32),
            pltpu.VMEM((16,), jnp.float32),
            pltpu.SMEM((16,), jnp.float32),
            pltpu.VMEM((RW,), jnp.int32),
            pltpu.VMEM((CR, H), jnp.float32),
            pltpu.VMEM((CR, H), jnp.float32),
            pltpu.VMEM((CP, H), jnp.float32),
            pltpu.VMEM((CP, H), jnp.float32),
            pltpu.SemaphoreType.DMA,
            pltpu.SemaphoreType.DMA,
            pltpu.SemaphoreType.DMA,
            pltpu.SemaphoreType.DMA,
        ],
    )
    def k(idx_hbm, wt_hbm, pos_hbm, chw_hbm, g_hbm, b_hbm, out_hbm,
          posv, gv, bv, wv, wsm, idxv, rb0, rb1,
          ob0, ob1, sem0, sem1, osem0, osem1):
        wid = lax.axis_index("s") * NC + lax.axis_index("c")
        row0 = wid * RW
        pair0 = wid * PW

        pltpu.sync_copy(pos_hbm, posv)
        pltpu.sync_copy(g_hbm, gv)
        pltpu.sync_copy(b_hbm, bv)
        pltpu.sync_copy(chw_hbm.at[pl.ds(0, 16)], wv)
        wvec = wv[...]
        for i in range(C):
            wsm[i] = wvec[i]

        pltpu.sync_copy(idx_hbm.at[pl.ds(row0, RW)], idxv)

        def gat(g, rb, sem):
            off = pl.multiple_of(g * CR, 8)
            return pltpu.make_async_copy(wt_hbm.at[idxv.at[pl.ds(off, CR)]],
                                         rb, sem)

        def owrite(g, ob, osem):
            off = pl.multiple_of(pair0 + g * CP, 8)
            return pltpu.make_async_copy(ob, out_hbm.at[pl.ds(off, CP)], osem)

        def process(g, rb, ob):
            gr0 = row0 + g * CR

            @functools.partial(plsc.parallel_loop, 0, CP // 2)
            def body_p(p2):
                for u in range(2):
                    p = p2 * 2 + u
                    base = p * C
                    acc = [jnp.zeros((16,), jnp.float32) for _ in range(4)]
                    offs = jnp.float32(0.0)
                    for c in range(C):
                        r = base + c
                        prow = lax.rem(gr0 + r, NPOS)
                        e = []
                        for kk in range(4):
                            e.append(rb[r, pl.ds(16 * kk, 16)]
                                     + posv[prow, pl.ds(16 * kk, 16)])
                        s = jnp.sum((e[0] + e[1]) + (e[2] + e[3]), axis=0)
                        q = jnp.sum((e[0] * e[0] + e[1] * e[1])
                                    + (e[2] * e[2] + e[3] * e[3]), axis=0)
                        mu = s * (1.0 / H)
                        var = q * (1.0 / H) - mu * mu + 1e-5
                        bi = lax.bitcast_convert_type(var, jnp.int32)
                        y = lax.bitcast_convert_type(
                            jnp.int32(0x5F3759DF) - (bi >> 1), jnp.float32)
                        for _ in range(4):
                            y = y * (1.5 - (0.5 * var) * (y * y))
                        a = wsm[c] * y
                        offs = offs + a * mu
                        for kk in range(4):
                            acc[kk] = acc[kk] + a * e[kk]
                    for kk in range(4):
                        ob[p, pl.ds(16 * kk, 16)] = (
                            (acc[kk] - offs) * gv[pl.ds(16 * kk, 16)]
                            + bv[pl.ds(16 * kk, 16)])

        gat(0, rb0, sem0).start()

        def body_m(h, carry):
            g0 = h * 2
            gat(g0, rb0, sem0).wait()
            gat(g0 + 1, rb1, sem1).start()

            @pl.when(g0 >= 2)
            def _():
                owrite(g0 - 2, ob0, osem0).wait()

            process(g0, rb0, ob0)
            owrite(g0, ob0, osem0).start()

            gat(g0 + 1, rb1, sem1).wait()

            @pl.when(g0 + 2 < NSTEP)
            def _():
                gat(g0 + 2, rb0, sem0).start()

            @pl.when(g0 >= 1)
            def _():
                owrite(g0 - 1, ob1, osem1).wait()

            process(g0 + 1, rb1, ob1)
            owrite(g0 + 1, ob1, osem1).start()
            return carry

        lax.fori_loop(0, NSTEP // 2, body_m, 0)
        owrite(NSTEP - 2, ob0, osem0).wait()
        owrite(NSTEP - 1, ob1, osem1).wait()

    return k(idx, word_table, pos_table, chw16, gamma, beta)


def _tc_softmax(chw128):
    def body(cb, ob):
        v = cb[...]
        lanei = lax.broadcasted_iota(jnp.int32, (1, 128), 1)
        valid = lanei < C
        vm = jnp.where(valid, v, -1e30)
        m = jnp.max(vm, axis=1, keepdims=True)
        e = jnp.where(valid, jnp.exp(vm - m), 0.0)
        ob[...] = e / jnp.sum(e, axis=1, keepdims=True)

    return pl.pallas_call(
        body,
        out_shape=jax.ShapeDtypeStruct((1, 128), jnp.float32),
    )(chw128)


def _tc_head(fused, w, b2d, g2d, be2d):
    blk = 512

    def body(xb, wb, bb, gb, beb, ob):
        y = jnp.dot(xb[...], wb[...], preferred_element_type=jnp.float32)
        y = y + bb[...]
        mu = jnp.mean(y, axis=1, keepdims=True)
        var = jnp.mean((y - mu) ** 2, axis=1, keepdims=True)
        ob[...] = (y - mu) * lax.rsqrt(var + 1e-5) * gb[...] + beb[...]

    return pl.pallas_call(
        body,
        grid=(P // blk,),
        in_specs=[
            pl.BlockSpec((blk, H), lambda i: (i, 0)),
            pl.BlockSpec((H, H), lambda i: (0, 0)),
            pl.BlockSpec((1, H), lambda i: (0, 0)),
            pl.BlockSpec((1, H), lambda i: (0, 0)),
            pl.BlockSpec((1, H), lambda i: (0, 0)),
        ],
        out_specs=pl.BlockSpec((blk, H), lambda i: (i, 0)),
        out_shape=jax.ShapeDtypeStruct((P, H), jnp.float32),
    )(fused, w, b2d, g2d, be2d)


def kernel(x, word_table, pos_table, gamma, beta, ch_w, W, b_lin, gamma2, beta2):
    idx = jnp.transpose(x, (0, 2, 1)).reshape(P * C)
    w128 = _tc_softmax(jnp.pad(ch_w, (0, 128 - C)).reshape(1, 128))
    fused = _sc_fused(idx, word_table, pos_table, w128.reshape(128),
                      gamma, beta)
    out = _tc_head(fused, W, b_lin.reshape(1, H), gamma2.reshape(1, H),
                   beta2.reshape(1, H))
    return out.reshape(B, S, H)
